# baseline (speedup 1.0000x reference)
import functools

import jax
import jax.numpy as jnp
from jax import lax
from jax.experimental import pallas as pl
from jax.experimental.pallas import tpu as pltpu
from jax.experimental.pallas import tpu_sc as plsc

NC = 2
NS = 16
NW = NC * NS
CH = 128
D = 128
G = 64

_mesh = plsc.VectorSubcoreMesh(core_axis_name="c", subcore_axis_name="s")


def _make_sc_degree(NP, EP, NCHW):
    R = NP // NS

    @functools.partial(
        pl.kernel,
        out_type=jax.ShapeDtypeStruct((NC, NP), jnp.float32),
        mesh=_mesh,
        scratch_types=[
            pltpu.VMEM((CH,), jnp.int32),
            pltpu.VMEM((CH,), jnp.float32),
            pltpu.VMEM_SHARED((NP,), jnp.float32),
        ],
    )
    def deg_kernel(dstp, ones_hbm, out, dv, onesv, acc):
        c = lax.axis_index("c")
        s = lax.axis_index("s")
        w = c * NS + s
        pltpu.sync_copy(ones_hbm.at[pl.ds(s * R, R)], acc.at[pl.ds(s * R, R)])
        pltpu.sync_copy(ones_hbm.at[pl.ds(0, CH)], onesv)
        plsc.subcore_barrier()
        base = w * CH * NCHW

        @pl.loop(0, NCHW)
        def _(j):
            pltpu.sync_copy(dstp.at[pl.ds(base + j * CH, CH)], dv)
            pltpu.sync_copy(onesv, acc.at[dv], add=True)

        plsc.subcore_barrier()
        pltpu.sync_copy(acc.at[pl.ds(s * R, R)], out.at[c, pl.ds(s * R, R)])

    return deg_kernel


def _make_sc_prop(NP, F0, F1):
    R = NP // NS

    NB = 2

    @functools.partial(
        pl.kernel,
        out_type=jax.ShapeDtypeStruct((NC, NP, D), jnp.float32),
        mesh=_mesh,
        scratch_types=[
            pltpu.VMEM((NB, CH), jnp.int32),
            pltpu.VMEM((NB, CH), jnp.int32),
            pltpu.VMEM((NB, CH, D), jnp.float32),
            pltpu.VMEM_SHARED((NP, D), jnp.float32),
            pltpu.SemaphoreType.DMA((NB,)),
            pltpu.SemaphoreType.DMA((NB,)),
        ],
    )
    def prop_kernel(hs, srcp, dstp, out, sv, dv, rv, acc, mg, ms):
        c = lax.axis_index("c")
        s = lax.axis_index("s")
        pltpu.sync_copy(hs.at[pl.ds(s * R, R)], acc.at[pl.ds(s * R, R)])
        plsc.subcore_barrier()
        nchw = jnp.where(c == 0, F0, F1)
        base = jnp.where(c == 0, s * F0, NS * F0 + s * F1) * CH

        def idx_load(cc, b):
            pltpu.sync_copy(srcp.at[pl.ds(base + cc * CH, CH)], sv.at[b])
            pltpu.sync_copy(dstp.at[pl.ds(base + cc * CH, CH)], dv.at[b])

        idx_load(0, 0)
        pltpu.async_copy(hs.at[sv.at[0]], rv.at[0], mg.at[0])

        @pl.loop(0, nchw, step=NB)
        def _(j):
            for b in range(NB):
                cc = j + b
                pltpu.make_async_copy(hs.at[sv.at[b]], rv.at[b], mg.at[b]).wait()
                pltpu.async_copy(rv.at[b], acc.at[dv.at[b]], ms.at[b], add=True)
                bb = (b + 1) % NB
                nxt = cc + 1

                @pl.when(nxt < nchw)
                def _():
                    @pl.when(cc >= 1)
                    def _():
                        pltpu.make_async_copy(
                            rv.at[bb], acc.at[dv.at[bb]], ms.at[bb]).wait()

                    idx_load(nxt, bb)
                    pltpu.async_copy(hs.at[sv.at[bb]], rv.at[bb], mg.at[bb])

        for b in range(NB):
            pltpu.make_async_copy(rv.at[b], acc.at[dv.at[b]], ms.at[b]).wait()

        plsc.subcore_barrier()
        pltpu.sync_copy(acc.at[pl.ds(s * R, R)], out.at[c, pl.ds(s * R, R)])

    return prop_kernel


def _tc_first(degp, x_p, W1, NP, BM):
    nblk = NP // BM

    def body(deg_ref, x_ref, w_ref, out_ref):
        dg = deg_ref[...]
        dinv = lax.rsqrt(dg[0] + dg[1] - 1.0)
        h = jnp.dot(x_ref[...], w_ref[...], preferred_element_type=jnp.float32)
        out_ref[...] = dinv[:, None] * h

    return pl.pallas_call(
        body,
        grid=(nblk,),
        in_specs=[
            pl.BlockSpec((NC, BM), lambda i: (0, i)),
            pl.BlockSpec((BM, D), lambda i: (i, 0)),
            pl.BlockSpec((D, D), lambda i: (0, 0)),
        ],
        out_specs=pl.BlockSpec((BM, D), lambda i: (i, 0)),
        out_shape=jax.ShapeDtypeStruct((NP, D), jnp.float32),
    )(degp, x_p, W1)


def _tc_mid(a, hs_prev, degp, b_prev, W, relu, NP, BM):
    nblk = NP // BM

    def body(a_ref, hs_ref, deg_ref, b_ref, w_ref, out_ref):
        dg = deg_ref[...]
        dinv = lax.rsqrt(dg[0] + dg[1] - 1.0)
        av = a_ref[...]
        t = dinv[:, None] * (av[0] + av[1] - hs_ref[...]) + b_ref[...]
        if relu:
            t = jnp.maximum(t, 0.0)
        out_ref[...] = dinv[:, None] * jnp.dot(
            t, w_ref[...], preferred_element_type=jnp.float32)

    return pl.pallas_call(
        body,
        grid=(nblk,),
        in_specs=[
            pl.BlockSpec((NC, BM, D), lambda i: (0, i, 0)),
            pl.BlockSpec((BM, D), lambda i: (i, 0)),
            pl.BlockSpec((NC, BM), lambda i: (0, i)),
            pl.BlockSpec((1, D), lambda i: (0, 0)),
            pl.BlockSpec((D, D), lambda i: (0, 0)),
        ],
        out_specs=pl.BlockSpec((BM, D), lambda i: (i, 0)),
        out_shape=jax.ShapeDtypeStruct((NP, D), jnp.float32),
    )(a, hs_prev, degp, b_prev, W)


def _tc_pool(a, hs_prev, degp, b_prev, batch_row, NP, BM):
    nblk = NP // BM

    def body(a_ref, hs_ref, deg_ref, b_ref, bat_ref, out_ref, acc_s, acc_c):
        i = pl.program_id(0)
        dg = deg_ref[...]
        dinv = lax.rsqrt(dg[0] + dg[1] - 1.0)
        av = a_ref[...]
        x3 = dinv[:, None] * (av[0] + av[1] - hs_ref[...]) + b_ref[...]
        gid = lax.broadcasted_iota(jnp.int32, (G, 1), 0)
        pt = (bat_ref[...] == gid).astype(jnp.float32)
        part = jnp.dot(pt, x3, preferred_element_type=jnp.float32)
        cnt = jnp.broadcast_to(jnp.sum(pt, axis=1, keepdims=True), (G, D))

        @pl.when(i == 0)
        def _():
            acc_s[...] = part
            acc_c[...] = cnt

        @pl.when(i > 0)
        def _():
            acc_s[...] += part
            acc_c[...] += cnt

        @pl.when(i == nblk - 1)
        def _():
            out_ref[...] = acc_s[...] / jnp.maximum(acc_c[...], 1.0)

    return pl.pallas_call(
        body,
        grid=(nblk,),
        in_specs=[
            pl.BlockSpec((NC, BM, D), lambda i: (0, i, 0)),
            pl.BlockSpec((BM, D), lambda i: (i, 0)),
            pl.BlockSpec((NC, BM), lambda i: (0, i)),
            pl.BlockSpec((1, D), lambda i: (0, 0)),
            pl.BlockSpec((1, BM), lambda i: (0, i)),
        ],
        out_specs=pl.BlockSpec((G, D), lambda i: (0, 0)),
        out_shape=jax.ShapeDtypeStruct((G, D), jnp.float32),
        scratch_shapes=[
            pltpu.VMEM((G, D), jnp.float32),
            pltpu.VMEM((G, D), jnp.float32),
        ],
    )(a, hs_prev, degp, b_prev, batch_row)


def kernel(x, edge_index, batch, W1, b1, W2, b2, W3, b3):
    N = x.shape[0]
    E = edge_index.shape[1]
    NP = (N // 2048 + 1) * 2048
    BM = NP // NS
    cpp = -(-E // (NS * CH))
    F0 = max(4, 4 * round(0.8 * cpp / 4))
    F1 = max(4, 4 * (-(-(cpp - F0) // 4)))
    EP = NS * (F0 + F1) * CH
    NCHW = (F0 + F1) // 2

    x_p = jnp.pad(x, ((0, NP - N), (0, 0)))
    pad_idx = jnp.full((EP - E,), N, jnp.int32)
    srcp = jnp.concatenate([edge_index[0], pad_idx])
    dstp = jnp.concatenate([edge_index[1], pad_idx])
    ones_h = jnp.ones((NP,), jnp.float32)
    batch_row = jnp.pad(batch, (0, NP - N), constant_values=G).reshape(1, NP)
    b1r, b2r, b3r = b1.reshape(1, D), b2.reshape(1, D), b3.reshape(1, D)

    degp = _make_sc_degree(NP, EP, NCHW)(dstp, ones_h)
    prop = _make_sc_prop(NP, F0, F1)

    hs1 = _tc_first(degp, x_p, W1, NP, BM)
    a1 = prop(hs1, srcp, dstp)
    hs2 = _tc_mid(a1, hs1, degp, b1r, W2, True, NP, BM)
    a2 = prop(hs2, srcp, dstp)
    hs3 = _tc_mid(a2, hs2, degp, b2r, W3, False, NP, BM)
    a3 = prop(hs3, srcp, dstp)
    return _tc_pool(a3, hs3, degp, b3r, batch_row, NP, BM)


# speedup vs baseline: 1.2106x; 1.2106x over previous
import functools

import jax
import jax.numpy as jnp
from jax import lax
from jax.experimental import pallas as pl
from jax.experimental.pallas import tpu as pltpu
from jax.experimental.pallas import tpu_sc as plsc

NC = 2
NS = 16
NW = NC * NS
CH = 128
D = 128
G = 64

_mesh = plsc.VectorSubcoreMesh(core_axis_name="c", subcore_axis_name="s")


def _make_sc_degree(NP, EP, NCHW):
    R = NP // NS

    @functools.partial(
        pl.kernel,
        out_type=jax.ShapeDtypeStruct((NC, NP), jnp.float32),
        mesh=_mesh,
        scratch_types=[
            pltpu.VMEM((CH,), jnp.int32),
            pltpu.VMEM((CH,), jnp.float32),
            pltpu.VMEM_SHARED((NP,), jnp.float32),
        ],
    )
    def deg_kernel(dstp, ones_hbm, out, dv, onesv, acc):
        c = lax.axis_index("c")
        s = lax.axis_index("s")
        w = c * NS + s
        pltpu.sync_copy(ones_hbm.at[pl.ds(s * R, R)], acc.at[pl.ds(s * R, R)])
        pltpu.sync_copy(ones_hbm.at[pl.ds(0, CH)], onesv)
        plsc.subcore_barrier()
        base = w * CH * NCHW

        @pl.loop(0, NCHW)
        def _(j):
            pltpu.sync_copy(dstp.at[pl.ds(base + j * CH, CH)], dv)
            pltpu.sync_copy(onesv, acc.at[dv], add=True)

        plsc.subcore_barrier()
        pltpu.sync_copy(acc.at[pl.ds(s * R, R)], out.at[c, pl.ds(s * R, R)])

    return deg_kernel


def _make_sc_prop(NP, F0, F1):
    R = NP // NS

    NB = 2

    @functools.partial(
        pl.kernel,
        out_type=jax.ShapeDtypeStruct((NC, NP, D), jnp.float32),
        mesh=_mesh,
        scratch_types=[
            pltpu.VMEM((NB, 2, CH), jnp.int32),
            pltpu.VMEM((NB, CH, D), jnp.float32),
            pltpu.VMEM_SHARED((NP, D), jnp.float32),
            pltpu.SemaphoreType.DMA((NB,)),
            pltpu.SemaphoreType.DMA((NB,)),
        ],
    )
    def prop_kernel(hs, eidx, out, iv, rv, acc, mg, mi):
        c = lax.axis_index("c")
        s = lax.axis_index("s")
        pltpu.sync_copy(hs.at[pl.ds(s * R, R)], acc.at[pl.ds(s * R, R)])
        plsc.subcore_barrier()
        nchw = jnp.where(c == 0, F0, F1)
        basech = jnp.where(c == 0, s * F0, NS * F0 + s * F1)

        pltpu.sync_copy(eidx.at[pl.ds(basech * 2, 2)], iv.at[0])
        pltpu.async_copy(hs.at[iv.at[0, 0]], rv.at[0], mg.at[0])
        pltpu.async_copy(eidx.at[pl.ds((basech + 1) * 2, 2)], iv.at[1],
                         mi.at[1])

        @pl.loop(0, nchw, step=NB)
        def _(j):
            for b in range(NB):
                cc = j + b
                ob = (b + 1) % NB

                @pl.when(cc + 1 < nchw)
                def _():
                    pltpu.make_async_copy(
                        eidx.at[pl.ds((basech + cc + 1) * 2, 2)], iv.at[ob],
                        mi.at[ob]).wait()
                    pltpu.async_copy(hs.at[iv.at[ob, 0]], rv.at[ob],
                                     mg.at[ob])

                pltpu.make_async_copy(hs.at[iv.at[b, 0]], rv.at[b],
                                      mg.at[b]).wait()
                pltpu.sync_copy(rv.at[b], acc.at[iv.at[b, 1]], add=True)

                @pl.when(cc + 2 < nchw)
                def _():
                    pltpu.async_copy(
                        eidx.at[pl.ds((basech + cc + 2) * 2, 2)], iv.at[b],
                        mi.at[b])

        plsc.subcore_barrier()
        pltpu.sync_copy(acc.at[pl.ds(s * R, R)], out.at[c, pl.ds(s * R, R)])

    return prop_kernel


def _tc_first(degp, x_p, W1, NP, BM):
    nblk = NP // BM

    def body(deg_ref, x_ref, w_ref, out_ref):
        dg = deg_ref[...]
        dinv = lax.rsqrt(dg[0] + dg[1] - 1.0)
        h = jnp.dot(x_ref[...], w_ref[...], preferred_element_type=jnp.float32)
        out_ref[...] = dinv[:, None] * h

    return pl.pallas_call(
        body,
        grid=(nblk,),
        in_specs=[
            pl.BlockSpec((NC, BM), lambda i: (0, i)),
            pl.BlockSpec((BM, D), lambda i: (i, 0)),
            pl.BlockSpec((D, D), lambda i: (0, 0)),
        ],
        out_specs=pl.BlockSpec((BM, D), lambda i: (i, 0)),
        out_shape=jax.ShapeDtypeStruct((NP, D), jnp.float32),
    )(degp, x_p, W1)


def _tc_mid(a, hs_prev, degp, b_prev, W, relu, NP, BM):
    nblk = NP // BM

    def body(a_ref, hs_ref, deg_ref, b_ref, w_ref, out_ref):
        dg = deg_ref[...]
        dinv = lax.rsqrt(dg[0] + dg[1] - 1.0)
        av = a_ref[...]
        t = dinv[:, None] * (av[0] + av[1] - hs_ref[...]) + b_ref[...]
        if relu:
            t = jnp.maximum(t, 0.0)
        out_ref[...] = dinv[:, None] * jnp.dot(
            t, w_ref[...], preferred_element_type=jnp.float32)

    return pl.pallas_call(
        body,
        grid=(nblk,),
        in_specs=[
            pl.BlockSpec((NC, BM, D), lambda i: (0, i, 0)),
            pl.BlockSpec((BM, D), lambda i: (i, 0)),
            pl.BlockSpec((NC, BM), lambda i: (0, i)),
            pl.BlockSpec((1, D), lambda i: (0, 0)),
            pl.BlockSpec((D, D), lambda i: (0, 0)),
        ],
        out_specs=pl.BlockSpec((BM, D), lambda i: (i, 0)),
        out_shape=jax.ShapeDtypeStruct((NP, D), jnp.float32),
    )(a, hs_prev, degp, b_prev, W)


def _tc_pool(a, hs_prev, degp, b_prev, batch_row, NP, BM):
    nblk = NP // BM

    def body(a_ref, hs_ref, deg_ref, b_ref, bat_ref, out_ref, acc_s, acc_c):
        i = pl.program_id(0)
        dg = deg_ref[...]
        dinv = lax.rsqrt(dg[0] + dg[1] - 1.0)
        av = a_ref[...]
        x3 = dinv[:, None] * (av[0] + av[1] - hs_ref[...]) + b_ref[...]
        gid = lax.broadcasted_iota(jnp.int32, (G, 1), 0)
        pt = (bat_ref[...] == gid).astype(jnp.float32)
        part = jnp.dot(pt, x3, preferred_element_type=jnp.float32)
        cnt = jnp.broadcast_to(jnp.sum(pt, axis=1, keepdims=True), (G, D))

        @pl.when(i == 0)
        def _():
            acc_s[...] = part
            acc_c[...] = cnt

        @pl.when(i > 0)
        def _():
            acc_s[...] += part
            acc_c[...] += cnt

        @pl.when(i == nblk - 1)
        def _():
            out_ref[...] = acc_s[...] / jnp.maximum(acc_c[...], 1.0)

    return pl.pallas_call(
        body,
        grid=(nblk,),
        in_specs=[
            pl.BlockSpec((NC, BM, D), lambda i: (0, i, 0)),
            pl.BlockSpec((BM, D), lambda i: (i, 0)),
            pl.BlockSpec((NC, BM), lambda i: (0, i)),
            pl.BlockSpec((1, D), lambda i: (0, 0)),
            pl.BlockSpec((1, BM), lambda i: (0, i)),
        ],
        out_specs=pl.BlockSpec((G, D), lambda i: (0, 0)),
        out_shape=jax.ShapeDtypeStruct((G, D), jnp.float32),
        scratch_shapes=[
            pltpu.VMEM((G, D), jnp.float32),
            pltpu.VMEM((G, D), jnp.float32),
        ],
    )(a, hs_prev, degp, b_prev, batch_row)


def kernel(x, edge_index, batch, W1, b1, W2, b2, W3, b3):
    N = x.shape[0]
    E = edge_index.shape[1]
    NP = (N // 2048 + 1) * 2048
    BM = NP // NS
    cpp = -(-E // (NS * CH))
    F0 = max(4, 4 * round(0.8 * cpp / 4))
    F1 = max(4, 4 * (-(-(cpp - F0) // 4)))
    EP = NS * (F0 + F1) * CH
    NCHW = (F0 + F1) // 2

    x_p = jnp.pad(x, ((0, NP - N), (0, 0)))
    pad_idx = jnp.full((EP - E,), N, jnp.int32)
    srcp = jnp.concatenate([edge_index[0], pad_idx])
    dstp = jnp.concatenate([edge_index[1], pad_idx])
    nch_total = EP // CH
    eidx = jnp.stack([srcp.reshape(nch_total, CH),
                      dstp.reshape(nch_total, CH)],
                     axis=1).reshape(nch_total * 2, CH)
    ones_h = jnp.ones((NP,), jnp.float32)
    batch_row = jnp.pad(batch, (0, NP - N), constant_values=G).reshape(1, NP)
    b1r, b2r, b3r = b1.reshape(1, D), b2.reshape(1, D), b3.reshape(1, D)

    degp = _make_sc_degree(NP, EP, NCHW)(dstp, ones_h)
    prop = _make_sc_prop(NP, F0, F1)

    hs1 = _tc_first(degp, x_p, W1, NP, BM)
    a1 = prop(hs1, eidx)
    hs2 = _tc_mid(a1, hs1, degp, b1r, W2, True, NP, BM)
    a2 = prop(hs2, eidx)
    hs3 = _tc_mid(a2, hs2, degp, b2r, W3, False, NP, BM)
    a3 = prop(hs3, eidx)
    return _tc_pool(a3, hs3, degp, b3r, batch_row, NP, BM)
